# Initial kernel scaffold; baseline (speedup 1.0000x reference)
import jax
import jax.numpy as jnp
from jax.experimental import pallas as pl


def kernel(enc_out, dec_out, W_enc, b_enc, W_dec, b_dec, W_out, b_out, targets, enc_lengths, target_lengths):
    raise NotImplementedError("write your pallas kernel here")


# speedup vs baseline: 7.7642x; 7.7642x over previous
import jax
import jax.numpy as jnp
from jax import lax
from jax.experimental import pallas as pl
from jax.experimental.pallas import tpu as pltpu

_B, _T, _U, _V = 4, 200, 100, 1024
_D_ENC, _D_DEC, _J = 144, 320, 320
_UPAD = 128
_TBLK = 8
_NT = _T // _TBLK
_PAIRS = _TBLK * _UPAD
_DROWS = 328
_NEG = -1e30

_INTERPRET = False


def _joint_kernel(enc_ref, dec_ref, wenc_ref, wdec_ref, woutT_ref, tgt_ref,
                  blank_ref, emit_ref):
    enc = enc_ref[0]
    dec = dec_ref[0]
    enc_pT = lax.dot_general(wenc_ref[...], enc, (((0,), (1,)), ((), ())),
                             preferred_element_type=jnp.float32)
    dec_pT = lax.dot_general(wdec_ref[...], dec, (((0,), (1,)), ((), ())),
                             preferred_element_type=jnp.float32)
    pieces = []
    for t in range(_TBLK):
        pieces.append(jnp.tanh(enc_pT[:, t:t + 1] + dec_pT))
    jointT = jnp.concatenate(pieces, axis=1)
    ones_row = jnp.ones((1, _PAIRS), jnp.float32)
    jointT_aug = jnp.concatenate([jointT, ones_row], axis=0)
    logitsT = lax.dot_general(woutT_ref[...], jointT_aug,
                              (((1,), (0,)), ((), ())),
                              preferred_element_type=jnp.float32)
    m = jnp.max(logitsT, axis=0, keepdims=True)
    ssum = jnp.sum(jnp.exp(logitsT - m), axis=0, keepdims=True)
    lse = m + jnp.log(ssum)
    blank = logitsT[0:1, :] - lse
    tgt = tgt_ref[0]
    vio = lax.broadcasted_iota(jnp.int32, (_V, _UPAD), 0)
    maskf = jnp.where(vio == tgt, 1.0, 0.0)
    mask_t = pltpu.repeat(maskf, _TBLK, axis=1)
    emit = jnp.sum(logitsT * mask_t, axis=0, keepdims=True) - lse
    blank_ref[0, 0] = blank
    emit_ref[0, 0] = emit


def _loss_kernel(blank_ref, emit_ref, tmask_ref, dstar_ref, out_ref,
                 bsk_ref, esk_ref):
    lane = lax.broadcasted_iota(jnp.int32, (1, _UPAD), 1)
    for b in range(_B):
        sb = blank_ref[b]
        se = emit_ref[b]
        for k in range(7):
            sh = 1 << k
            bit = (lane & sh) != 0
            sb = jnp.where(bit, pltpu.roll(sb, sh, axis=0), sb)
            se = jnp.where(bit, pltpu.roll(se, sh, axis=0), se)
        bsk_ref[b] = sb
        esk_ref[b] = se

    lane_b = lax.broadcasted_iota(jnp.int32, (_B, _UPAD), 1)
    tmask = tmask_ref[...]
    dstar = dstar_ref[...]
    alpha0 = jnp.where(lane_b == 0, 0.0, _NEG)
    capA0 = jnp.sum(jnp.where(dstar == 0, alpha0 * tmask, 0.0),
                    axis=1, keepdims=True)
    capB0 = jnp.zeros((_B, 1), jnp.float32)

    def body(d, carry):
        alpha, capA, capB = carry
        bs = bsk_ref[:, pl.ds(d - 1, 1), :].reshape(_B, _UPAD)
        es = esk_ref[:, pl.ds(d - 1, 1), :].reshape(_B, _UPAD)
        horiz = pltpu.roll(alpha + es, 1, axis=1)
        horiz = jnp.where(lane_b == 0, _NEG, horiz)
        alpha_new = jnp.logaddexp(alpha + bs, horiz)
        valA = jnp.sum(alpha_new * tmask, axis=1, keepdims=True)
        valB = jnp.sum(bs * tmask, axis=1, keepdims=True)
        eqA = dstar[:, 0:1] == d
        eqB = dstar[:, 0:1] == (d - 1)
        capA = jnp.where(eqA, valA, capA)
        capB = jnp.where(eqB, valB, capB)
        return alpha_new, capA, capB

    _, capA, capB = lax.fori_loop(1, _T + _U + 1, body,
                                  (alpha0, capA0, capB0))
    tot = jnp.sum(capA + capB, axis=0, keepdims=True)
    out_ref[...] = tot * (-1.0 / _B)


def kernel(enc_out, dec_out, W_enc, b_enc, W_dec, b_dec, W_out, b_out,
           targets, enc_lengths, target_lengths):
    f32 = jnp.float32
    dec_aug = jnp.concatenate(
        [dec_out, jnp.ones((_B, _U + 1, 1), f32)], axis=2)
    dec_aug = jnp.pad(dec_aug, ((0, 0), (0, _UPAD - (_U + 1)), (0, 0)))
    W_dec_aug = jnp.concatenate([W_dec, (b_enc + b_dec)[None, :]], axis=0)
    W_outT_aug = jnp.concatenate([W_out.T, b_out[:, None]], axis=1)
    tgt3 = jnp.pad(targets.astype(jnp.int32),
                   ((0, 0), (0, _UPAD - _U)))[:, None, :]

    blank4, emit4 = pl.pallas_call(
        _joint_kernel,
        grid=(_B, _NT),
        in_specs=[
            pl.BlockSpec((1, _TBLK, _D_ENC), lambda b, i: (b, i, 0)),
            pl.BlockSpec((1, _UPAD, _D_DEC + 1), lambda b, i: (b, 0, 0)),
            pl.BlockSpec((_D_ENC, _J), lambda b, i: (0, 0)),
            pl.BlockSpec((_D_DEC + 1, _J), lambda b, i: (0, 0)),
            pl.BlockSpec((_V, _J + 1), lambda b, i: (0, 0)),
            pl.BlockSpec((1, 1, _UPAD), lambda b, i: (b, 0, 0)),
        ],
        out_specs=[
            pl.BlockSpec((1, 1, 1, _PAIRS), lambda b, i: (b, i, 0, 0)),
            pl.BlockSpec((1, 1, 1, _PAIRS), lambda b, i: (b, i, 0, 0)),
        ],
        out_shape=[
            jax.ShapeDtypeStruct((_B, _NT, 1, _PAIRS), f32),
            jax.ShapeDtypeStruct((_B, _NT, 1, _PAIRS), f32),
        ],
        compiler_params=pltpu.CompilerParams(
            dimension_semantics=("parallel", "arbitrary"),
        ),
        interpret=_INTERPRET,
    )(enc_out, dec_aug, W_enc, W_dec_aug, W_outT_aug, tgt3)

    blank = blank4.reshape(_B, _T, _UPAD)
    emit = emit4.reshape(_B, _T, _UPAD)
    padrows = jnp.full((_B, _DROWS - _T, _UPAD), _NEG, f32)
    blank_pad = jnp.concatenate([blank, padrows], axis=1)
    emit_pad = jnp.concatenate([emit, padrows], axis=1)

    tl = target_lengths.astype(jnp.int32)
    el = enc_lengths.astype(jnp.int32)
    tmask = (jnp.arange(_UPAD, dtype=jnp.int32)[None, :]
             == tl[:, None]).astype(f32)
    dstar = jnp.broadcast_to((el - 1 + tl)[:, None], (_B, _UPAD))

    out = pl.pallas_call(
        _loss_kernel,
        grid=(1,),
        in_specs=[
            pl.BlockSpec((_B, _DROWS, _UPAD), lambda i: (0, 0, 0)),
            pl.BlockSpec((_B, _DROWS, _UPAD), lambda i: (0, 0, 0)),
            pl.BlockSpec((_B, _UPAD), lambda i: (0, 0)),
            pl.BlockSpec((_B, _UPAD), lambda i: (0, 0)),
        ],
        out_specs=pl.BlockSpec((1, 1), lambda i: (0, 0)),
        out_shape=jax.ShapeDtypeStruct((1, 1), f32),
        scratch_shapes=[
            pltpu.VMEM((_B, _DROWS, _UPAD), f32),
            pltpu.VMEM((_B, _DROWS, _UPAD), f32),
        ],
        interpret=_INTERPRET,
    )(blank_pad, emit_pad, tmask, dstar)
    return out[0, 0]
